# baseline (speedup 1.0000x reference)
import functools

import jax
import jax.numpy as jnp
from jax import lax
from jax.experimental import pallas as pl
from jax.experimental.pallas import tpu as pltpu, tpu_sc as plsc

N = 10000
NP = 10240
E = 320000
E2 = 327680
D_IN = 128
D_HID = 128
D_OUT = 40
D4P = 128
EPS = 1e-5

NC = 2
NS = 16
NW = NC * NS
EPW = E2 // NW
CH = 64
NCHUNK = EPW // CH
RPS = NP // NS
ZR = 32

_vmesh = plsc.VectorSubcoreMesh(core_axis_name="c", subcore_axis_name="s",
                                num_cores=NC, num_subcores=NS)


DEG_NIB = 8


@functools.partial(
    pl.kernel,
    out_type=jax.ShapeDtypeStruct((NC, NP), jnp.float32),
    mesh=_vmesh,
    scratch_types=(
        [pltpu.VMEM((CH,), jnp.int32)] * DEG_NIB
        + [pltpu.VMEM((CH,), jnp.float32),
           pltpu.VMEM((NP // NS,), jnp.float32),
           pltpu.VMEM_SHARED((NP,), jnp.float32)]
        + [pltpu.SemaphoreType.DMA] * DEG_NIB
    ),
)
def _sc_degree(dst_hbm, deg_out, *refs):
    didx = refs[0:DEG_NIB]
    ones_v = refs[DEG_NIB]
    zbuf = refs[DEG_NIB + 1]
    dacc = refs[DEG_NIB + 2]
    sem_di = refs[DEG_NIB + 3:2 * DEG_NIB + 3]

    c = lax.axis_index("c")
    s = lax.axis_index("s")
    w = s * NC + c

    zeros16 = jnp.zeros((16,), jnp.float32)
    ones16 = jnp.ones((16,), jnp.float32)

    for b in range(DEG_NIB):
        base = w * EPW + b * CH
        pltpu.async_copy(dst_hbm.at[pl.ds(base, CH)], didx[b], sem_di[b])

    def fill_ones(i, _):
        ones_v[pl.ds(i * 16, 16)] = ones16
        return 0

    lax.fori_loop(0, CH // 16, fill_ones, 0)

    def fill_zero(i, _):
        zbuf[pl.ds(i * 16, 16)] = zeros16
        return 0

    lax.fori_loop(0, RPS // 16, fill_zero, 0)
    pltpu.sync_copy(zbuf, dacc.at[pl.ds(s * RPS, RPS)])
    plsc.subcore_barrier()

    def blk(g, _):
        for b in range(DEG_NIB):
            i = g * DEG_NIB + b
            pltpu.make_async_copy(dst_hbm.at[pl.ds(0, CH)], didx[b],
                                  sem_di[b]).wait()
            pltpu.sync_copy(ones_v, dacc.at[didx[b]], add=True)
            nxt = i + DEG_NIB

            @pl.when(nxt < NCHUNK)
            def _():
                nbase = w * EPW + nxt * CH
                pltpu.async_copy(dst_hbm.at[pl.ds(nbase, CH)], didx[b],
                                 sem_di[b])

        return 0

    lax.fori_loop(0, NCHUNK // DEG_NIB, blk, 0)
    plsc.subcore_barrier()

    pltpu.sync_copy(dacc.at[pl.ds(s * RPS, RPS)], zbuf)
    pltpu.sync_copy(zbuf, deg_out.at[c, pl.ds(s * RPS, RPS)])


NBUF = 5
GA = 4
NIB = 10


def _make_sc_scatter(D):
    @functools.partial(
        pl.kernel,
        out_type=jax.ShapeDtypeStruct((NC, NP, D), jnp.float32),
        mesh=_vmesh,
        scratch_types=(
            [pltpu.VMEM((CH,), jnp.int32)] * (2 * NIB)
            + [pltpu.VMEM((CH, D), jnp.float32)] * NBUF
            + [pltpu.VMEM((ZR, D), jnp.float32),
               pltpu.VMEM_SHARED((NP, D), jnp.float32)]
            + [pltpu.SemaphoreType.DMA] * (2 * NBUF + 2 * NIB)
        ),
    )
    def k(y_hbm, src_hbm, dst_hbm, out_hbm, *refs):
        sidx = refs[0:NIB]
        didx = refs[NIB:2 * NIB]
        rows = refs[2 * NIB:2 * NIB + NBUF]
        buf_v = refs[2 * NIB + NBUF]
        acc_sh = refs[2 * NIB + NBUF + 1]
        o = 2 * NIB + NBUF + 2
        sem_g = refs[o:o + NBUF]
        sem_sc = refs[o + NBUF:o + 2 * NBUF]
        sem_si = refs[o + 2 * NBUF:o + 2 * NBUF + NIB]
        sem_di = refs[o + 2 * NBUF + NIB:o + 2 * NBUF + 2 * NIB]

        c = lax.axis_index("c")
        s = lax.axis_index("s")
        w = s * NC + c

        zeros16 = jnp.zeros((16,), jnp.float32)

        def zero_body(i, _):
            def cols(j, _):
                buf_v[i, pl.ds(j * 16, 16)] = zeros16
                return 0

            lax.fori_loop(0, D // 16, cols, 0)
            return 0

        lax.fori_loop(0, ZR, zero_body, 0)

        for b in range(NIB):
            base = w * EPW + b * CH
            pltpu.async_copy(src_hbm.at[pl.ds(base, CH)], sidx[b], sem_si[b])
            pltpu.async_copy(dst_hbm.at[pl.ds(base, CH)], didx[b], sem_di[b])

        for b in range(RPS // ZR):
            pltpu.sync_copy(buf_v, acc_sh.at[pl.ds(s * RPS + b * ZR, ZR)])
        plsc.subcore_barrier()

        for b in range(GA):
            pltpu.make_async_copy(src_hbm.at[pl.ds(0, CH)], sidx[b],
                                  sem_si[b]).wait()
            pltpu.async_copy(y_hbm.at[sidx[b]], rows[b], sem_g[b])

        def blk(g, _):
            for b in range(NIB):
                i = g * NIB + b
                b5 = b % NBUF
                bf = (b + GA) % NBUF
                bi = (b + GA) % NIB
                bd = (b - 1) % NIB

                pltpu.make_async_copy(y_hbm.at[sidx[b]], rows[b5],
                                      sem_g[b5]).wait()
                pltpu.make_async_copy(dst_hbm.at[pl.ds(0, CH)], didx[b],
                                      sem_di[b]).wait()
                pltpu.async_copy(rows[b5], acc_sh.at[didx[b]], sem_sc[b5],
                                 add=True)
                nxt_i = i + NIB

                @pl.when(nxt_i < NCHUNK)
                def _():
                    nbase = w * EPW + nxt_i * CH
                    pltpu.async_copy(src_hbm.at[pl.ds(nbase, CH)], sidx[b],
                                     sem_si[b])

                nxt_g = i + GA

                @pl.when((i >= 1) & (nxt_g < NCHUNK))
                def _():
                    pltpu.make_async_copy(rows[bf], acc_sh.at[didx[bd]],
                                          sem_sc[bf]).wait()

                nxt_d = i - 1 + NIB

                @pl.when((i >= 1) & (nxt_d < NCHUNK))
                def _():
                    dbase = w * EPW + nxt_d * CH
                    pltpu.async_copy(dst_hbm.at[pl.ds(dbase, CH)], didx[bd],
                                     sem_di[bd])

                @pl.when(nxt_g < NCHUNK)
                def _():
                    pltpu.make_async_copy(src_hbm.at[pl.ds(0, CH)], sidx[bi],
                                          sem_si[bi]).wait()
                    pltpu.async_copy(y_hbm.at[sidx[bi]], rows[bf], sem_g[bf])

            return 0

        lax.fori_loop(0, NCHUNK // NIB, blk, 0)

        for b in range(NBUF):
            pltpu.make_async_copy(rows[b], acc_sh.at[didx[0]],
                                  sem_sc[b]).wait()
        plsc.subcore_barrier()

        for b in range(RPS // ZR):
            r0 = s * RPS + b * ZR
            pltpu.sync_copy(acc_sh.at[pl.ds(r0, ZR)], buf_v)
            pltpu.sync_copy(buf_v, out_hbm.at[c, pl.ds(r0, ZR)])

    return k


_sc_scatter_hid = _make_sc_scatter(D_HID)


BN = 2048


def _tc_prep(x, W1, degp):
    def body(x_ref, w_ref, degp_ref, dis_ref, y_ref):
        deg = jnp.sum(degp_ref[...], axis=0) + 1.0
        dis = lax.rsqrt(deg)[:, None]
        dis_ref[...] = dis
        y_ref[...] = jnp.dot(x_ref[...], w_ref[...],
                             preferred_element_type=jnp.float32) * dis

    return pl.pallas_call(
        body,
        grid=(NP // BN,),
        in_specs=[
            pl.BlockSpec((BN, D_IN), lambda i: (i, 0)),
            pl.BlockSpec((D_IN, D_HID), lambda i: (0, 0)),
            pl.BlockSpec((NC, BN), lambda i: (0, i)),
        ],
        out_specs=[
            pl.BlockSpec((BN, 1), lambda i: (i, 0)),
            pl.BlockSpec((BN, D_HID), lambda i: (i, 0)),
        ],
        out_shape=[
            jax.ShapeDtypeStruct((NP, 1), jnp.float32),
            jax.ShapeDtypeStruct((NP, D_HID), jnp.float32),
        ],
    )(x, W1, degp)


def _tc_layer(agg, y, dis, sc, sh, Wn):
    Do = Wn.shape[1]

    def body(agg_ref, y_ref, dis_ref, sc_ref, sh_ref, w_ref, o_ref):
        t = (agg_ref[0] + agg_ref[1] + y_ref[...]) * dis_ref[...]
        h = jnp.maximum(t * sc_ref[...] + sh_ref[...], 0.0)
        o_ref[...] = jnp.dot(h, w_ref[...],
                             preferred_element_type=jnp.float32) * dis_ref[...]

    return pl.pallas_call(
        body,
        grid=(NP // BN,),
        in_specs=[
            pl.BlockSpec((NC, BN, D_HID), lambda i: (0, i, 0)),
            pl.BlockSpec((BN, D_HID), lambda i: (i, 0)),
            pl.BlockSpec((BN, 1), lambda i: (i, 0)),
            pl.BlockSpec((1, D_HID), lambda i: (0, 0)),
            pl.BlockSpec((1, D_HID), lambda i: (0, 0)),
            pl.BlockSpec((D_HID, Do), lambda i: (0, 0)),
        ],
        out_specs=pl.BlockSpec((BN, Do), lambda i: (i, 0)),
        out_shape=jax.ShapeDtypeStruct((NP, Do), jnp.float32),
    )(agg, y, dis, sc, sh, Wn)


def _tc_final(agg, y, dis, b4p):
    def body(agg_ref, y_ref, dis_ref, b_ref, o_ref):
        o = (agg_ref[0] + agg_ref[1] + y_ref[...]) * dis_ref[...] + b_ref[...]
        col = lax.broadcasted_iota(jnp.int32, (BN, D4P), 1)
        valid = col < D_OUT
        m = jnp.max(jnp.where(valid, o, -jnp.inf), axis=1, keepdims=True)
        ex = jnp.where(valid, jnp.exp(o - m), 0.0)
        lse = jnp.log(jnp.sum(ex, axis=1, keepdims=True))
        o_ref[...] = (o - m - lse)[:, :D_OUT]

    return pl.pallas_call(
        body,
        grid=(NP // BN,),
        in_specs=[
            pl.BlockSpec((NC, BN, D4P), lambda i: (0, i, 0)),
            pl.BlockSpec((BN, D4P), lambda i: (i, 0)),
            pl.BlockSpec((BN, 1), lambda i: (i, 0)),
            pl.BlockSpec((1, D4P), lambda i: (0, 0)),
        ],
        out_specs=pl.BlockSpec((BN, D_OUT), lambda i: (i, 0)),
        out_shape=jax.ShapeDtypeStruct((NP, D_OUT), jnp.float32),
    )(agg, y, dis, b4p)


def kernel(x, edge_index, W1, b1, W2, b2, W3, b3, W4, b4,
           g1, be1, g2, be2, g3, be3):
    src = edge_index[0]
    dst = edge_index[1]
    pad_idx = (N + jnp.arange(E2 - E, dtype=src.dtype) % (NP - N))
    src = jnp.concatenate([src, pad_idx])
    dst = jnp.concatenate([dst, pad_idx])

    x_pad = jnp.pad(x, ((0, NP - N), (0, 0)))
    inv = 1.0 / jnp.sqrt(1.0 + EPS)
    s1 = (g1 * inv)[None, :]
    s2 = (g2 * inv)[None, :]
    s3 = (g3 * inv)[None, :]
    sh1 = (b1 * s1[0] + be1)[None, :]
    sh2 = (b2 * s2[0] + be2)[None, :]
    sh3 = (b3 * s3[0] + be3)[None, :]
    W4p = jnp.pad(W4, ((0, 0), (0, D4P - D_OUT)))
    b4p = jnp.pad(b4, (0, D4P - D_OUT))[None, :]

    degp = _sc_degree(dst)
    dis, y1 = _tc_prep(x_pad, W1, degp)

    agg1 = _sc_scatter_hid(y1, src, dst)
    y2 = _tc_layer(agg1, y1, dis, s1, sh1, W2)
    agg2 = _sc_scatter_hid(y2, src, dst)
    y3 = _tc_layer(agg2, y2, dis, s2, sh2, W3)
    agg3 = _sc_scatter_hid(y3, src, dst)
    y4 = _tc_layer(agg3, y3, dis, s3, sh3, W4p)
    agg4 = _sc_scatter_hid(y4, src, dst)
    out = _tc_final(agg4, y4, dis, b4p)
    return out[:N]


# speedup vs baseline: 1.0609x; 1.0609x over previous
import functools

import jax
import jax.numpy as jnp
from jax import lax
from jax.experimental import pallas as pl
from jax.experimental.pallas import tpu as pltpu, tpu_sc as plsc

N = 10000
NP = 10240
E = 320000
E2 = 327680
D_IN = 128
D_HID = 128
D_OUT = 40
D4P = 128
EPS = 1e-5

NC = 2
NS = 16
NW = NC * NS
EPW = E2 // NW
CH = 64
NCHUNK = EPW // CH
RPS = NP // NS
ZR = 32

_vmesh = plsc.VectorSubcoreMesh(core_axis_name="c", subcore_axis_name="s",
                                num_cores=NC, num_subcores=NS)


DEG_NIB = 8


@functools.partial(
    pl.kernel,
    out_type=jax.ShapeDtypeStruct((NC, NP), jnp.float32),
    mesh=_vmesh,
    scratch_types=(
        [pltpu.VMEM((CH,), jnp.int32)] * DEG_NIB
        + [pltpu.VMEM((CH,), jnp.float32),
           pltpu.VMEM((NP // NS,), jnp.float32),
           pltpu.VMEM_SHARED((NP,), jnp.float32)]
        + [pltpu.SemaphoreType.DMA] * DEG_NIB
    ),
)
def _sc_degree(dst_hbm, deg_out, *refs):
    didx = refs[0:DEG_NIB]
    ones_v = refs[DEG_NIB]
    zbuf = refs[DEG_NIB + 1]
    dacc = refs[DEG_NIB + 2]
    sem_di = refs[DEG_NIB + 3:2 * DEG_NIB + 3]

    c = lax.axis_index("c")
    s = lax.axis_index("s")
    w = s * NC + c

    zeros16 = jnp.zeros((16,), jnp.float32)
    ones16 = jnp.ones((16,), jnp.float32)

    for b in range(DEG_NIB):
        base = w * EPW + b * CH
        pltpu.async_copy(dst_hbm.at[pl.ds(base, CH)], didx[b], sem_di[b])

    def fill_ones(i, _):
        ones_v[pl.ds(i * 16, 16)] = ones16
        return 0

    lax.fori_loop(0, CH // 16, fill_ones, 0)

    def fill_zero(i, _):
        zbuf[pl.ds(i * 16, 16)] = zeros16
        return 0

    lax.fori_loop(0, RPS // 16, fill_zero, 0)
    pltpu.sync_copy(zbuf, dacc.at[pl.ds(s * RPS, RPS)])
    plsc.subcore_barrier()

    def blk(g, _):
        for b in range(DEG_NIB):
            i = g * DEG_NIB + b
            pltpu.make_async_copy(dst_hbm.at[pl.ds(0, CH)], didx[b],
                                  sem_di[b]).wait()
            pltpu.sync_copy(ones_v, dacc.at[didx[b]], add=True)
            nxt = i + DEG_NIB

            @pl.when(nxt < NCHUNK)
            def _():
                nbase = w * EPW + nxt * CH
                pltpu.async_copy(dst_hbm.at[pl.ds(nbase, CH)], didx[b],
                                 sem_di[b])

        return 0

    lax.fori_loop(0, NCHUNK // DEG_NIB, blk, 0)
    plsc.subcore_barrier()

    pltpu.sync_copy(dacc.at[pl.ds(s * RPS, RPS)], zbuf)
    pltpu.sync_copy(zbuf, deg_out.at[c, pl.ds(s * RPS, RPS)])


NBUF = 4
NIB = 8


def _make_sc_scatter(D):
    @functools.partial(
        pl.kernel,
        out_type=jax.ShapeDtypeStruct((NC, NP, D), jnp.float32),
        mesh=_vmesh,
        scratch_types=(
            [pltpu.VMEM((CH,), jnp.int32)] * (2 * NIB)
            + [pltpu.VMEM((CH, D), jnp.float32)] * NBUF
            + [pltpu.VMEM((ZR, D), jnp.float32)] * 2
            + [pltpu.VMEM_SHARED((NP, D), jnp.float32)]
            + [pltpu.SemaphoreType.DMA] * (NBUF + 2 * NIB + 2)
        ),
    )
    def k(y_hbm, src_hbm, dst_hbm, out_hbm, *refs):
        sidx = refs[0:NIB]
        didx = refs[NIB:2 * NIB]
        rows = refs[2 * NIB:2 * NIB + NBUF]
        bufs = refs[2 * NIB + NBUF:2 * NIB + NBUF + 2]
        buf_v = bufs[0]
        acc_sh = refs[2 * NIB + NBUF + 2]
        o = 2 * NIB + NBUF + 3
        sems = refs[o:o + NBUF]
        sem_si = refs[o + NBUF:o + NBUF + NIB]
        sem_di = refs[o + NBUF + NIB:o + NBUF + 2 * NIB]
        sem_w = refs[o + NBUF + 2 * NIB:o + NBUF + 2 * NIB + 2]

        c = lax.axis_index("c")
        s = lax.axis_index("s")
        w = s * NC + c

        zeros16 = jnp.zeros((16,), jnp.float32)

        def zero_body(i, _):
            def cols(j, _):
                buf_v[i, pl.ds(j * 16, 16)] = zeros16
                return 0

            lax.fori_loop(0, D // 16, cols, 0)
            return 0

        lax.fori_loop(0, ZR, zero_body, 0)

        for b in range(NIB):
            base = w * EPW + b * CH
            pltpu.async_copy(src_hbm.at[pl.ds(base, CH)], sidx[b], sem_si[b])
            pltpu.async_copy(dst_hbm.at[pl.ds(base, CH)], didx[b], sem_di[b])

        for b in range(RPS // ZR):
            pltpu.sync_copy(buf_v, acc_sh.at[pl.ds(s * RPS + b * ZR, ZR)])
        plsc.subcore_barrier()

        for b in range(NBUF):
            pltpu.make_async_copy(src_hbm.at[pl.ds(0, CH)], sidx[b],
                                  sem_si[b]).wait()
            pltpu.async_copy(y_hbm.at[sidx[b]], rows[b], sems[b])

        def blk(g, _):
            for b in range(NIB):
                i = g * NIB + b
                b4 = b % NBUF
                bp = (b + NBUF) % NIB
                pltpu.make_async_copy(y_hbm.at[sidx[b]], rows[b4],
                                      sems[b4]).wait()
                pltpu.make_async_copy(dst_hbm.at[pl.ds(0, CH)], didx[b],
                                      sem_di[b]).wait()
                pltpu.sync_copy(rows[b4], acc_sh.at[didx[b]], add=True)

                nxt_i = i + NIB

                @pl.when(nxt_i < NCHUNK)
                def _():
                    nbase = w * EPW + nxt_i * CH
                    pltpu.async_copy(src_hbm.at[pl.ds(nbase, CH)], sidx[b],
                                     sem_si[b])
                    pltpu.async_copy(dst_hbm.at[pl.ds(nbase, CH)], didx[b],
                                     sem_di[b])

                nxt_g = i + NBUF

                @pl.when(nxt_g < NCHUNK)
                def _():
                    pltpu.make_async_copy(src_hbm.at[pl.ds(0, CH)], sidx[bp],
                                          sem_si[bp]).wait()
                    pltpu.async_copy(y_hbm.at[sidx[bp]], rows[b4], sems[b4])

            return 0

        lax.fori_loop(0, NCHUNK // NIB, blk, 0)
        plsc.subcore_barrier()

        for t in range(RPS // ZR):
            bb = t % 2
            r0 = s * RPS + t * ZR
            if t >= 2:
                pltpu.make_async_copy(bufs[bb], out_hbm.at[c, pl.ds(0, ZR)],
                                      sem_w[bb]).wait()
            pltpu.sync_copy(acc_sh.at[pl.ds(r0, ZR)], bufs[bb])
            pltpu.async_copy(bufs[bb], out_hbm.at[c, pl.ds(r0, ZR)],
                             sem_w[bb])
        for bb in range(2):
            pltpu.make_async_copy(bufs[bb], out_hbm.at[c, pl.ds(0, ZR)],
                                  sem_w[bb]).wait()

    return k


_sc_scatter_hid = _make_sc_scatter(D_HID)


BN = 2048


def _tc_prep(x, W1, degp):
    def body(x_ref, w_ref, degp_ref, dis_ref, y_ref):
        deg = jnp.sum(degp_ref[...], axis=0) + 1.0
        dis = lax.rsqrt(deg)[:, None]
        dis_ref[...] = dis
        y_ref[...] = jnp.dot(x_ref[...], w_ref[...],
                             preferred_element_type=jnp.float32) * dis

    return pl.pallas_call(
        body,
        grid=(NP // BN,),
        in_specs=[
            pl.BlockSpec((BN, D_IN), lambda i: (i, 0)),
            pl.BlockSpec((D_IN, D_HID), lambda i: (0, 0)),
            pl.BlockSpec((NC, BN), lambda i: (0, i)),
        ],
        out_specs=[
            pl.BlockSpec((BN, 1), lambda i: (i, 0)),
            pl.BlockSpec((BN, D_HID), lambda i: (i, 0)),
        ],
        out_shape=[
            jax.ShapeDtypeStruct((NP, 1), jnp.float32),
            jax.ShapeDtypeStruct((NP, D_HID), jnp.float32),
        ],
    )(x, W1, degp)


def _tc_layer(agg, y, dis, sc, sh, Wn):
    Do = Wn.shape[1]

    def body(agg_ref, y_ref, dis_ref, sc_ref, sh_ref, w_ref, o_ref):
        t = (agg_ref[0] + agg_ref[1] + y_ref[...]) * dis_ref[...]
        h = jnp.maximum(t * sc_ref[...] + sh_ref[...], 0.0)
        o_ref[...] = jnp.dot(h, w_ref[...],
                             preferred_element_type=jnp.float32) * dis_ref[...]

    return pl.pallas_call(
        body,
        grid=(NP // BN,),
        in_specs=[
            pl.BlockSpec((NC, BN, D_HID), lambda i: (0, i, 0)),
            pl.BlockSpec((BN, D_HID), lambda i: (i, 0)),
            pl.BlockSpec((BN, 1), lambda i: (i, 0)),
            pl.BlockSpec((1, D_HID), lambda i: (0, 0)),
            pl.BlockSpec((1, D_HID), lambda i: (0, 0)),
            pl.BlockSpec((D_HID, Do), lambda i: (0, 0)),
        ],
        out_specs=pl.BlockSpec((BN, Do), lambda i: (i, 0)),
        out_shape=jax.ShapeDtypeStruct((NP, Do), jnp.float32),
    )(agg, y, dis, sc, sh, Wn)


def _tc_final(agg, y, dis, b4p):
    def body(agg_ref, y_ref, dis_ref, b_ref, o_ref):
        o = (agg_ref[0] + agg_ref[1] + y_ref[...]) * dis_ref[...] + b_ref[...]
        col = lax.broadcasted_iota(jnp.int32, (BN, D4P), 1)
        valid = col < D_OUT
        m = jnp.max(jnp.where(valid, o, -jnp.inf), axis=1, keepdims=True)
        ex = jnp.where(valid, jnp.exp(o - m), 0.0)
        lse = jnp.log(jnp.sum(ex, axis=1, keepdims=True))
        o_ref[...] = (o - m - lse)[:, :D_OUT]

    return pl.pallas_call(
        body,
        grid=(NP // BN,),
        in_specs=[
            pl.BlockSpec((NC, BN, D4P), lambda i: (0, i, 0)),
            pl.BlockSpec((BN, D4P), lambda i: (i, 0)),
            pl.BlockSpec((BN, 1), lambda i: (i, 0)),
            pl.BlockSpec((1, D4P), lambda i: (0, 0)),
        ],
        out_specs=pl.BlockSpec((BN, D_OUT), lambda i: (i, 0)),
        out_shape=jax.ShapeDtypeStruct((NP, D_OUT), jnp.float32),
    )(agg, y, dis, b4p)


def kernel(x, edge_index, W1, b1, W2, b2, W3, b3, W4, b4,
           g1, be1, g2, be2, g3, be3):
    src = edge_index[0]
    dst = edge_index[1]
    pad_idx = (N + jnp.arange(E2 - E, dtype=src.dtype) % (NP - N))
    src = jnp.concatenate([src, pad_idx])
    dst = jnp.concatenate([dst, pad_idx])

    x_pad = jnp.pad(x, ((0, NP - N), (0, 0)))
    inv = 1.0 / jnp.sqrt(1.0 + EPS)
    s1 = (g1 * inv)[None, :]
    s2 = (g2 * inv)[None, :]
    s3 = (g3 * inv)[None, :]
    sh1 = (b1 * s1[0] + be1)[None, :]
    sh2 = (b2 * s2[0] + be2)[None, :]
    sh3 = (b3 * s3[0] + be3)[None, :]
    W4p = jnp.pad(W4, ((0, 0), (0, D4P - D_OUT)))
    b4p = jnp.pad(b4, (0, D4P - D_OUT))[None, :]

    degp = _sc_degree(dst)
    dis, y1 = _tc_prep(x_pad, W1, degp)

    agg1 = _sc_scatter_hid(y1, src, dst)
    y2 = _tc_layer(agg1, y1, dis, s1, sh1, W2)
    agg2 = _sc_scatter_hid(y2, src, dst)
    y3 = _tc_layer(agg2, y2, dis, s2, sh2, W3)
    agg3 = _sc_scatter_hid(y3, src, dst)
    y4 = _tc_layer(agg3, y3, dis, s3, sh3, W4p)
    agg4 = _sc_scatter_hid(y4, src, dst)
    out = _tc_final(agg4, y4, dis, b4p)
    return out[:N]
